# baseline (speedup 1.0000x reference)
import functools

import jax
import jax.numpy as jnp
from jax import lax
from jax.experimental import pallas as pl
from jax.experimental.pallas import tpu as pltpu
from jax.experimental.pallas import tpu_sc as plsc

_N_ACTIONS = 1000
_BATCH = 16384
_NUM_CORES = 2
_NUM_SUBCORES = 16
_NW = _NUM_CORES * _NUM_SUBCORES
_BPW = _BATCH // _NW
_CHUNK = 128
_NCH = _BPW // _CHUNK
_LANES = 16

_mesh = plsc.VectorSubcoreMesh(core_axis_name="c", subcore_axis_name="s")


@functools.partial(
    pl.kernel,
    out_type=jax.ShapeDtypeStruct((_NW, _NCH, _CHUNK), jnp.float32),
    mesh=_mesh,
    scratch_types=[
        pltpu.VMEM((_NCH, _CHUNK), jnp.int32),
        pltpu.VMEM((_NCH, _CHUNK), jnp.int32),
        pltpu.VMEM((_NCH, _CHUNK), jnp.float32),
        pltpu.SemaphoreType.DMA,
    ],
)
def _gather_kernel(flat_hbm, idx_hbm, out_hbm, idx_v, flat_v, val_v, sem):
    wid = lax.axis_index("s") * _NUM_CORES + lax.axis_index("c")
    base = wid * _BPW

    pltpu.sync_copy(idx_hbm.at[wid], idx_v)

    for j in range(_NCH):
        for t in range(_CHUNK // _LANES):
            row0 = base + j * _CHUNK + t * _LANES
            rows = row0 + lax.iota(jnp.int32, _LANES)
            cols = idx_v[j, pl.ds(t * _LANES, _LANES)]
            flat_v[j, pl.ds(t * _LANES, _LANES)] = rows * _N_ACTIONS + cols

    copies = [
        pltpu.async_copy(flat_hbm.at[flat_v.at[j]], val_v.at[j], sem)
        for j in range(_NCH)
    ]
    for c in copies:
        c.wait()

    pltpu.sync_copy(val_v, out_hbm.at[wid])


def kernel(full_output, indices):
    flat = full_output.reshape(-1)
    idx = indices.astype(jnp.int32).reshape(_NW, _NCH, _CHUNK)
    out = _gather_kernel(flat, idx)
    return out.reshape(-1)


# speedup vs baseline: 1.4761x; 1.4761x over previous
import functools

import jax
import jax.numpy as jnp
from jax import lax
from jax.experimental import pallas as pl
from jax.experimental.pallas import tpu as pltpu
from jax.experimental.pallas import tpu_sc as plsc

_N_ACTIONS = 1000
_BATCH = 16384
_NW = 32
_RPW = _BATCH // _NW
_NBLK = _BATCH // 8
_BPW = _NBLK // _NW
_CB = 4
_NCH = _BPW // _CB
_L = 16

_mesh = plsc.VectorSubcoreMesh(core_axis_name="c", subcore_axis_name="s")


@functools.partial(
    pl.kernel,
    out_type=jax.ShapeDtypeStruct((_BATCH,), jnp.float32),
    mesh=_mesh,
    scratch_types=[
        pltpu.VMEM((_RPW,), jnp.int32),
        pltpu.VMEM((_CB, 8, _N_ACTIONS), jnp.float32),
        pltpu.VMEM((_CB, 8, _N_ACTIONS), jnp.float32),
        pltpu.VMEM((_RPW,), jnp.float32),
        pltpu.SemaphoreType.DMA,
        pltpu.SemaphoreType.DMA,
    ],
    compiler_params=pltpu.CompilerParams(needs_layout_passes=False),
)
def _gather_kernel(mat_hbm, idx_hbm, out_hbm,
                   idx_v, buf_a, buf_b, out_v, sem_a, sem_b):
    wid = lax.axis_index("s") * 2 + lax.axis_index("c")
    base = wid * _RPW
    blk0 = wid * _BPW

    pltpu.sync_copy(idx_hbm.at[pl.ds(base, _RPW)], idx_v)

    bufs = (buf_a, buf_b)
    sems = (sem_a, sem_b)
    copies = [None, None]
    rpc = _CB * 8

    def start(c):
        b = c % 2
        copies[b] = pltpu.async_copy(
            mat_hbm.at[pl.ds(blk0 + c * _CB, _CB)], bufs[b], sems[b])

    def extract(c):
        b = c % 2
        copies[b].wait()
        buf = bufs[b]
        for s in range(rpc // _L):
            off = c * rpc + s * _L
            cols = idx_v[pl.ds(off, _L)]
            local = s * _L + lax.iota(jnp.int32, _L)
            b16 = lax.shift_right_logical(local, 3)
            r16 = local & 7
            out_v[pl.ds(off, _L)] = plsc.load_gather(buf, [b16, r16, cols])

    start(0)
    for c in range(_NCH):
        if c + 1 < _NCH:
            start(c + 1)
        extract(c)

    pltpu.sync_copy(out_v, out_hbm.at[pl.ds(base, _RPW)])


def kernel(full_output, indices):
    mat = full_output.reshape(_NBLK, 8, _N_ACTIONS)
    idx = indices.astype(jnp.int32)
    return _gather_kernel(mat, idx)
